# baseline (speedup 1.0000x reference)
import functools

import jax
import jax.numpy as jnp
from jax import lax
from jax.experimental import pallas as pl
from jax.experimental.pallas import tpu as pltpu
from jax.experimental.pallas import tpu_sc as plsc

D = 768
HD = D // 2
H = 1536
NR = 7
N = 2048
T = 256
G_MAX = 22
P_RT = G_MAX * T
G_TOT = G_MAX + N // T
P_TOT = G_TOT * T
NW = 32
CHUNK = N // NW


def _pack(xb):
    lo = lax.bitcast_convert_type(xb[:, :HD], jnp.uint16).astype(jnp.uint32)
    hi = lax.bitcast_convert_type(xb[:, HD:], jnp.uint16).astype(jnp.uint32)
    return lax.bitcast_convert_type((hi << 16) | lo, jnp.float32)


def _unpack(pk):
    u = lax.bitcast_convert_type(pk, jnp.uint32)
    lo = lax.bitcast_convert_type((u & 0xFFFF).astype(jnp.uint16),
                                  jnp.bfloat16)
    hi = lax.bitcast_convert_type((u >> 16).astype(jnp.uint16), jnp.bfloat16)
    return jnp.concatenate([lo, hi], axis=1)


def _router_body(x_ref, wr_ref, pos0_ref, pos1_ref, w0_ref, w1_ref, te_ref,
                 xpk_ref):
    xb = x_ref[...]
    xpk_ref[...] = _pack(xb.astype(jnp.bfloat16))

    logits = jnp.dot(xb, wr_ref[...], preferred_element_type=jnp.float32)
    iot = lax.broadcasted_iota(jnp.int32, (N, NR), 1)
    m1 = jnp.max(logits, axis=1, keepdims=True)
    i1 = jnp.min(jnp.where(logits == m1, iot, NR), axis=1, keepdims=True)
    masked = jnp.where(iot == i1, -1e30, logits)
    m2 = jnp.max(masked, axis=1, keepdims=True)
    i2 = jnp.min(jnp.where(masked == m2, iot, NR), axis=1, keepdims=True)
    e2 = jnp.exp(m2 - m1)
    wa = 1.0 / (1.0 + e2)
    wb = 1.0 - wa

    A = (iot == i1).astype(jnp.float32) + (iot == i2).astype(jnp.float32)

    r128 = lax.broadcasted_iota(jnp.int32, (128, 128), 0)
    c128 = lax.broadcasted_iota(jnp.int32, (128, 128), 1)
    tri128 = (c128 < r128).astype(jnp.float32)
    nblk = N // 128
    r16 = lax.broadcasted_iota(jnp.int32, (nblk, nblk), 0)
    c16 = lax.broadcasted_iota(jnp.int32, (nblk, nblk), 1)
    tri16 = (c16 < r16).astype(jnp.float32)

    withins = []
    sums = []
    for b in range(nblk):
        Ab = A[b * 128:(b + 1) * 128, :]
        withins.append(jnp.dot(tri128, Ab, preferred_element_type=jnp.float32))
        sums.append(jnp.sum(Ab, axis=0, keepdims=True))
    S = jnp.concatenate(sums, axis=0)
    blockpref = jnp.dot(tri16, S, preferred_element_type=jnp.float32)
    prefix = jnp.concatenate(
        [withins[b] + blockpref[b:b + 1, :] for b in range(nblk)], axis=0)

    counts = jnp.sum(S, axis=0, keepdims=True)
    ntiles = jnp.floor((counts + (T - 1)) * (1.0 / T))
    ends = jnp.dot(ntiles, (lax.broadcasted_iota(jnp.int32, (NR, NR), 0) <=
                            lax.broadcasted_iota(jnp.int32, (NR, NR), 1)
                            ).astype(jnp.float32),
                   preferred_element_type=jnp.float32)
    base = (ends - ntiles) * float(T)

    onehot1 = (iot == i1).astype(jnp.float32)
    onehot2 = (iot == i2).astype(jnp.float32)
    pos0 = jnp.sum(onehot1 * (base + prefix), axis=1, keepdims=True)
    pos1 = jnp.sum(onehot2 * (base + prefix), axis=1, keepdims=True)

    pos0_ref[...] = pos0.astype(jnp.int32)
    pos1_ref[...] = pos1.astype(jnp.int32)
    w0_ref[...] = wa
    w1_ref[...] = wb

    tiot = lax.broadcasted_iota(jnp.int32, (1, 64), 1).astype(jnp.float32)
    te = jnp.zeros((1, 64), jnp.float32)
    for e in range(NR):
        te = te + (tiot >= ends[0:1, e:e + 1]).astype(jnp.float32)
    te_ref[...] = te.astype(jnp.int32)


@jax.jit
def _router(xf, Wr):
    return pl.pallas_call(
        _router_body,
        in_specs=[pl.BlockSpec((N, D), lambda: (0, 0)),
                  pl.BlockSpec((D, NR), lambda: (0, 0))],
        out_specs=[pl.BlockSpec((N, 1), lambda: (0, 0)),
                   pl.BlockSpec((N, 1), lambda: (0, 0)),
                   pl.BlockSpec((N, 1), lambda: (0, 0)),
                   pl.BlockSpec((N, 1), lambda: (0, 0)),
                   pl.BlockSpec((1, 64), lambda: (0, 0)),
                   pl.BlockSpec((N, HD), lambda: (0, 0))],
        out_shape=[jax.ShapeDtypeStruct((N, 1), jnp.int32),
                   jax.ShapeDtypeStruct((N, 1), jnp.int32),
                   jax.ShapeDtypeStruct((N, 1), jnp.float32),
                   jax.ShapeDtypeStruct((N, 1), jnp.float32),
                   jax.ShapeDtypeStruct((1, 64), jnp.int32),
                   jax.ShapeDtypeStruct((N, HD), jnp.float32)],
    )(xf, Wr)


@functools.lru_cache(maxsize=None)
def _make_dispatch():
    mesh = plsc.VectorSubcoreMesh(core_axis_name="c", subcore_axis_name="s")

    @functools.partial(
        pl.kernel, mesh=mesh,
        out_type=jax.ShapeDtypeStruct((P_RT, HD), jnp.float32),
        scratch_types=[
            pltpu.VMEM((CHUNK, HD), jnp.float32),
            pltpu.VMEM((CHUNK,), jnp.int32),
            pltpu.VMEM((CHUNK,), jnp.int32),
            pltpu.SemaphoreType.DMA,
        ],
    )
    def dispatch(x_hbm, p0_hbm, p1_hbm, xg_hbm, xv, p0v, p1v, sem):
        wid = lax.axis_index("s") * 2 + lax.axis_index("c")
        base = wid * CHUNK
        pltpu.sync_copy(x_hbm.at[pl.ds(base, CHUNK)], xv)
        pltpu.sync_copy(p0_hbm.at[pl.ds(base, CHUNK)], p0v)
        pltpu.sync_copy(p1_hbm.at[pl.ds(base, CHUNK)], p1v)
        pltpu.async_copy(xv, xg_hbm.at[p0v], sem).wait()
        pltpu.async_copy(xv, xg_hbm.at[p1v], sem).wait()

    return dispatch


BTS = 512


def _final_body(ysh_ref, sel0_ref, sel1_ref, wa_ref, wb_ref, out_ref):
    ysh = _unpack(ysh_ref[...]).astype(jnp.float32)
    sel0 = _unpack(sel0_ref[...]).astype(jnp.float32)
    sel1 = _unpack(sel1_ref[...]).astype(jnp.float32)
    out_ref[...] = ysh + wa_ref[...] * sel0 + wb_ref[...] * sel1


@jax.jit
def _final(yg, sel0, sel1, w0, w1):
    sh_base = P_RT // BTS
    return pl.pallas_call(
        _final_body,
        grid=(N // BTS,),
        in_specs=[pl.BlockSpec((BTS, HD), lambda t: (sh_base + t, 0)),
                  pl.BlockSpec((BTS, HD), lambda t: (t, 0)),
                  pl.BlockSpec((BTS, HD), lambda t: (t, 0)),
                  pl.BlockSpec((BTS, 1), lambda t: (t, 0)),
                  pl.BlockSpec((BTS, 1), lambda t: (t, 0))],
        out_specs=pl.BlockSpec((BTS, D), lambda t: (t, 0)),
        out_shape=jax.ShapeDtypeStruct((N, D), jnp.float32),
    )(yg, sel0, sel1, w0, w1)


def _swiglu_pk(xb, w1_ref, w2_ref, w3_ref):
    a1 = jnp.dot(xb, w1_ref[0].astype(jnp.bfloat16),
                 preferred_element_type=jnp.float32)
    a2 = jnp.dot(xb, w2_ref[0].astype(jnp.bfloat16),
                 preferred_element_type=jnp.float32)
    act = a1 * (1.0 / (1.0 + jnp.exp(-a1))) * a2
    y = jnp.dot(act.astype(jnp.bfloat16),
                w3_ref[0].astype(jnp.bfloat16),
                preferred_element_type=jnp.float32)
    return _pack(y.astype(jnp.bfloat16))


def _grouped_body(te_ref, xg_ref, xpk_ref, w1_ref, w2_ref, w3_ref,
                  ws1_ref, ws2_ref, ws3_ref, yg_ref):
    t = pl.program_id(0)

    @pl.when(te_ref[t] < NR)
    def _():
        yg_ref[...] = _swiglu_pk(_unpack(xg_ref[...]), w1_ref, w2_ref, w3_ref)

    @pl.when(t >= G_MAX)
    def _():
        yg_ref[...] = _swiglu_pk(_unpack(xpk_ref[...]),
                                 ws1_ref, ws2_ref, ws3_ref)


@jax.jit
def _grouped(te, xg, xpk, W1, W2, W3, Ws1, Ws2, Ws3):
    grid_spec = pltpu.PrefetchScalarGridSpec(
        num_scalar_prefetch=1,
        grid=(G_TOT,),
        in_specs=[
            pl.BlockSpec((T, HD),
                         lambda t, te: (jnp.minimum(t, G_MAX - 1), 0)),
            pl.BlockSpec((T, HD),
                         lambda t, te: (jnp.maximum(t - G_MAX, 0), 0)),
            pl.BlockSpec((1, D, H),
                         lambda t, te: (jnp.minimum(te[t], NR - 1), 0, 0)),
            pl.BlockSpec((1, D, H),
                         lambda t, te: (jnp.minimum(te[t], NR - 1), 0, 0)),
            pl.BlockSpec((1, H, D),
                         lambda t, te: (jnp.minimum(te[t], NR - 1), 0, 0)),
            pl.BlockSpec((1, D, H), lambda t, te: (0, 0, 0)),
            pl.BlockSpec((1, D, H), lambda t, te: (0, 0, 0)),
            pl.BlockSpec((1, H, D), lambda t, te: (0, 0, 0)),
        ],
        out_specs=pl.BlockSpec((T, HD), lambda t, te: (t, 0)),
    )
    return pl.pallas_call(
        _grouped_body,
        grid_spec=grid_spec,
        out_shape=jax.ShapeDtypeStruct((P_TOT, HD), jnp.float32),
    )(te, xg, xpk, W1, W2, W3, Ws1, Ws2, Ws3)


@functools.lru_cache(maxsize=None)
def _make_gather2():
    mesh = plsc.VectorSubcoreMesh(core_axis_name="c", subcore_axis_name="s")

    @functools.partial(
        pl.kernel, mesh=mesh,
        out_type=[jax.ShapeDtypeStruct((N, HD), jnp.float32),
                  jax.ShapeDtypeStruct((N, HD), jnp.float32)],
        scratch_types=[
            pltpu.VMEM((CHUNK, HD), jnp.float32),
            pltpu.VMEM((CHUNK, HD), jnp.float32),
            pltpu.VMEM((CHUNK,), jnp.int32),
            pltpu.VMEM((CHUNK,), jnp.int32),
            pltpu.SemaphoreType.DMA,
            pltpu.SemaphoreType.DMA,
        ],
    )
    def gather2(yg_hbm, p0_hbm, p1_hbm, sel0_hbm, sel1_hbm,
                bufa, bufb, p0v, p1v, sema, semb):
        wid = lax.axis_index("s") * 2 + lax.axis_index("c")
        base = wid * CHUNK
        pltpu.sync_copy(p0_hbm.at[pl.ds(base, CHUNK)], p0v)
        pltpu.sync_copy(p1_hbm.at[pl.ds(base, CHUNK)], p1v)
        cpa = pltpu.async_copy(yg_hbm.at[p0v], bufa, sema)
        cpb = pltpu.async_copy(yg_hbm.at[p1v], bufb, semb)
        cpa.wait()
        pltpu.sync_copy(bufa, sel0_hbm.at[pl.ds(base, CHUNK)])
        cpb.wait()
        pltpu.sync_copy(bufb, sel1_hbm.at[pl.ds(base, CHUNK)])

    return gather2


def kernel(x, Wr, W1, W2, W3, Ws1, Ws2, Ws3):
    orig_shape = x.shape
    xf = x.reshape(-1, orig_shape[-1])
    pos0, pos1, w0, w1, te, xpk = _router(xf, Wr)
    p0f = pos0.reshape(N)
    p1f = pos1.reshape(N)
    xg = _make_dispatch()(xpk, p0f, p1f)
    yg = _grouped(te.reshape(64), xg, xpk, W1, W2, W3, Ws1, Ws2, Ws3)
    sel0, sel1 = _make_gather2()(yg, p0f, p1f)
    out = _final(yg, sel0, sel1, w0, w1)
    return out.reshape(orig_shape)


# speedup vs baseline: 1.0508x; 1.0508x over previous
import functools

import jax
import jax.numpy as jnp
from jax import lax
from jax.experimental import pallas as pl
from jax.experimental.pallas import tpu as pltpu
from jax.experimental.pallas import tpu_sc as plsc

D = 768
HD = D // 2
H = 1536
NR = 7
N = 2048
T = 256
G_MAX = 22
P_RT = G_MAX * T
NW = 32
CHUNK = N // NW


def _pack(xb):
    lo = lax.bitcast_convert_type(xb[:, :HD], jnp.uint16).astype(jnp.uint32)
    hi = lax.bitcast_convert_type(xb[:, HD:], jnp.uint16).astype(jnp.uint32)
    return lax.bitcast_convert_type((hi << 16) | lo, jnp.float32)


def _unpack(pk):
    u = lax.bitcast_convert_type(pk, jnp.uint32)
    lo = lax.bitcast_convert_type((u & 0xFFFF).astype(jnp.uint16),
                                  jnp.bfloat16)
    hi = lax.bitcast_convert_type((u >> 16).astype(jnp.uint16), jnp.bfloat16)
    return jnp.concatenate([lo, hi], axis=1)


def _router_body(x_ref, wr_ref, pos0_ref, pos1_ref, w0_ref, w1_ref, te_ref,
                 xpk_ref):
    xb = x_ref[...]
    xpk_ref[...] = _pack(xb.astype(jnp.bfloat16))

    logits = jnp.dot(xb, wr_ref[...], preferred_element_type=jnp.float32)
    iot = lax.broadcasted_iota(jnp.int32, (N, NR), 1)
    m1 = jnp.max(logits, axis=1, keepdims=True)
    i1 = jnp.min(jnp.where(logits == m1, iot, NR), axis=1, keepdims=True)
    masked = jnp.where(iot == i1, -1e30, logits)
    m2 = jnp.max(masked, axis=1, keepdims=True)
    i2 = jnp.min(jnp.where(masked == m2, iot, NR), axis=1, keepdims=True)
    e2 = jnp.exp(m2 - m1)
    wa = 1.0 / (1.0 + e2)
    wb = 1.0 - wa

    A = (iot == i1).astype(jnp.float32) + (iot == i2).astype(jnp.float32)

    r128 = lax.broadcasted_iota(jnp.int32, (128, 128), 0)
    c128 = lax.broadcasted_iota(jnp.int32, (128, 128), 1)
    tri128 = (c128 < r128).astype(jnp.float32)
    nblk = N // 128
    r16 = lax.broadcasted_iota(jnp.int32, (nblk, nblk), 0)
    c16 = lax.broadcasted_iota(jnp.int32, (nblk, nblk), 1)
    tri16 = (c16 < r16).astype(jnp.float32)

    withins = []
    sums = []
    for b in range(nblk):
        Ab = A[b * 128:(b + 1) * 128, :]
        withins.append(jnp.dot(tri128, Ab, preferred_element_type=jnp.float32))
        sums.append(jnp.sum(Ab, axis=0, keepdims=True))
    S = jnp.concatenate(sums, axis=0)
    blockpref = jnp.dot(tri16, S, preferred_element_type=jnp.float32)
    prefix = jnp.concatenate(
        [withins[b] + blockpref[b:b + 1, :] for b in range(nblk)], axis=0)

    counts = jnp.sum(S, axis=0, keepdims=True)
    ntiles = jnp.floor((counts + (T - 1)) * (1.0 / T))
    ends = jnp.dot(ntiles, (lax.broadcasted_iota(jnp.int32, (NR, NR), 0) <=
                            lax.broadcasted_iota(jnp.int32, (NR, NR), 1)
                            ).astype(jnp.float32),
                   preferred_element_type=jnp.float32)
    base = (ends - ntiles) * float(T)

    onehot1 = (iot == i1).astype(jnp.float32)
    onehot2 = (iot == i2).astype(jnp.float32)
    pos0 = jnp.sum(onehot1 * (base + prefix), axis=1, keepdims=True)
    pos1 = jnp.sum(onehot2 * (base + prefix), axis=1, keepdims=True)

    pos0_ref[...] = pos0.astype(jnp.int32)
    pos1_ref[...] = pos1.astype(jnp.int32)
    w0_ref[...] = wa
    w1_ref[...] = wb

    tiot = lax.broadcasted_iota(jnp.int32, (1, 64), 1).astype(jnp.float32)
    te = jnp.zeros((1, 64), jnp.float32)
    for e in range(NR):
        te = te + (tiot >= ends[0:1, e:e + 1]).astype(jnp.float32)
    te_ref[...] = te.astype(jnp.int32)


@jax.jit
def _router(xf, Wr):
    return pl.pallas_call(
        _router_body,
        in_specs=[pl.BlockSpec((N, D), lambda: (0, 0)),
                  pl.BlockSpec((D, NR), lambda: (0, 0))],
        out_specs=[pl.BlockSpec((N, 1), lambda: (0, 0)),
                   pl.BlockSpec((N, 1), lambda: (0, 0)),
                   pl.BlockSpec((N, 1), lambda: (0, 0)),
                   pl.BlockSpec((N, 1), lambda: (0, 0)),
                   pl.BlockSpec((1, 64), lambda: (0, 0)),
                   pl.BlockSpec((N, HD), lambda: (0, 0))],
        out_shape=[jax.ShapeDtypeStruct((N, 1), jnp.int32),
                   jax.ShapeDtypeStruct((N, 1), jnp.int32),
                   jax.ShapeDtypeStruct((N, 1), jnp.float32),
                   jax.ShapeDtypeStruct((N, 1), jnp.float32),
                   jax.ShapeDtypeStruct((1, 64), jnp.int32),
                   jax.ShapeDtypeStruct((N, HD), jnp.float32)],
    )(xf, Wr)


@functools.lru_cache(maxsize=None)
def _make_dispatch():
    mesh = plsc.VectorSubcoreMesh(core_axis_name="c", subcore_axis_name="s")

    @functools.partial(
        pl.kernel, mesh=mesh,
        out_type=jax.ShapeDtypeStruct((P_RT, HD), jnp.float32),
        scratch_types=[
            pltpu.VMEM((CHUNK, HD), jnp.float32),
            pltpu.VMEM((CHUNK,), jnp.int32),
            pltpu.VMEM((CHUNK,), jnp.int32),
            pltpu.SemaphoreType.DMA,
        ],
    )
    def dispatch(x_hbm, p0_hbm, p1_hbm, xg_hbm, xv, p0v, p1v, sem):
        wid = lax.axis_index("s") * 2 + lax.axis_index("c")
        base = wid * CHUNK
        pltpu.sync_copy(x_hbm.at[pl.ds(base, CHUNK)], xv)
        pltpu.sync_copy(p0_hbm.at[pl.ds(base, CHUNK)], p0v)
        pltpu.sync_copy(p1_hbm.at[pl.ds(base, CHUNK)], p1v)
        pltpu.async_copy(xv, xg_hbm.at[p0v], sem).wait()
        pltpu.async_copy(xv, xg_hbm.at[p1v], sem).wait()

    return dispatch


BTS = 512


def _final_body(x_ref, w1_ref, w2_ref, w3_ref, sel0_ref, sel1_ref,
                wa_ref, wb_ref, out_ref):
    xb = _unpack(x_ref[...])
    a1 = jnp.dot(xb, w1_ref[0].astype(jnp.bfloat16),
                 preferred_element_type=jnp.float32)
    a2 = jnp.dot(xb, w2_ref[0].astype(jnp.bfloat16),
                 preferred_element_type=jnp.float32)
    act = a1 * (1.0 / (1.0 + jnp.exp(-a1))) * a2
    ysh = jnp.dot(act.astype(jnp.bfloat16),
                  w3_ref[0].astype(jnp.bfloat16),
                  preferred_element_type=jnp.float32)
    sel0 = _unpack(sel0_ref[...]).astype(jnp.float32)
    sel1 = _unpack(sel1_ref[...]).astype(jnp.float32)
    out_ref[...] = ysh + wa_ref[...] * sel0 + wb_ref[...] * sel1


@jax.jit
def _final(xf, Ws1, Ws2, Ws3, sel0, sel1, w0, w1):
    return pl.pallas_call(
        _final_body,
        grid=(N // BTS,),
        in_specs=[pl.BlockSpec((BTS, HD), lambda t: (t, 0)),
                  pl.BlockSpec((1, D, H), lambda t: (0, 0, 0)),
                  pl.BlockSpec((1, D, H), lambda t: (0, 0, 0)),
                  pl.BlockSpec((1, H, D), lambda t: (0, 0, 0)),
                  pl.BlockSpec((BTS, HD), lambda t: (t, 0)),
                  pl.BlockSpec((BTS, HD), lambda t: (t, 0)),
                  pl.BlockSpec((BTS, 1), lambda t: (t, 0)),
                  pl.BlockSpec((BTS, 1), lambda t: (t, 0))],
        out_specs=pl.BlockSpec((BTS, D), lambda t: (t, 0)),
        out_shape=jax.ShapeDtypeStruct((N, D), jnp.float32),
    )(xf, Ws1, Ws2, Ws3, sel0, sel1, w0, w1)


def _grouped_body(te_ref, xg_ref, w1_ref, w2_ref, w3_ref, yg_ref):
    t = pl.program_id(0)

    @pl.when(te_ref[t] < NR)
    def _():
        xb = _unpack(xg_ref[...])
        a1 = jnp.dot(xb, w1_ref[0].astype(jnp.bfloat16),
                     preferred_element_type=jnp.float32)
        a2 = jnp.dot(xb, w2_ref[0].astype(jnp.bfloat16),
                     preferred_element_type=jnp.float32)
        act = a1 * (1.0 / (1.0 + jnp.exp(-a1))) * a2
        y = jnp.dot(act.astype(jnp.bfloat16),
                    w3_ref[0].astype(jnp.bfloat16),
                    preferred_element_type=jnp.float32)
        yg_ref[...] = _pack(y.astype(jnp.bfloat16))


@jax.jit
def _grouped(te, xg, W1, W2, W3):
    grid_spec = pltpu.PrefetchScalarGridSpec(
        num_scalar_prefetch=1,
        grid=(G_MAX,),
        in_specs=[
            pl.BlockSpec((T, HD), lambda t, te: (t, 0)),
            pl.BlockSpec((1, D, H),
                         lambda t, te: (jnp.minimum(te[t], NR - 1), 0, 0)),
            pl.BlockSpec((1, D, H),
                         lambda t, te: (jnp.minimum(te[t], NR - 1), 0, 0)),
            pl.BlockSpec((1, H, D),
                         lambda t, te: (jnp.minimum(te[t], NR - 1), 0, 0)),
        ],
        out_specs=pl.BlockSpec((T, HD), lambda t, te: (t, 0)),
    )
    return pl.pallas_call(
        _grouped_body,
        grid_spec=grid_spec,
        out_shape=jax.ShapeDtypeStruct((P_RT, HD), jnp.float32),
    )(te, xg, W1, W2, W3)


@functools.lru_cache(maxsize=None)
def _make_gather2():
    mesh = plsc.VectorSubcoreMesh(core_axis_name="c", subcore_axis_name="s")

    @functools.partial(
        pl.kernel, mesh=mesh,
        out_type=[jax.ShapeDtypeStruct((N, HD), jnp.float32),
                  jax.ShapeDtypeStruct((N, HD), jnp.float32)],
        scratch_types=[
            pltpu.VMEM((CHUNK, HD), jnp.float32),
            pltpu.VMEM((CHUNK, HD), jnp.float32),
            pltpu.VMEM((CHUNK,), jnp.int32),
            pltpu.VMEM((CHUNK,), jnp.int32),
            pltpu.SemaphoreType.DMA,
            pltpu.SemaphoreType.DMA,
        ],
    )
    def gather2(yg_hbm, p0_hbm, p1_hbm, sel0_hbm, sel1_hbm,
                bufa, bufb, p0v, p1v, sema, semb):
        wid = lax.axis_index("s") * 2 + lax.axis_index("c")
        base = wid * CHUNK
        pltpu.sync_copy(p0_hbm.at[pl.ds(base, CHUNK)], p0v)
        pltpu.sync_copy(p1_hbm.at[pl.ds(base, CHUNK)], p1v)
        cpa = pltpu.async_copy(yg_hbm.at[p0v], bufa, sema)
        cpb = pltpu.async_copy(yg_hbm.at[p1v], bufb, semb)
        cpa.wait()
        pltpu.sync_copy(bufa, sel0_hbm.at[pl.ds(base, CHUNK)])
        cpb.wait()
        pltpu.sync_copy(bufb, sel1_hbm.at[pl.ds(base, CHUNK)])

    return gather2


def kernel(x, Wr, W1, W2, W3, Ws1, Ws2, Ws3):
    orig_shape = x.shape
    xf = x.reshape(-1, orig_shape[-1])
    pos0, pos1, w0, w1, te, xpk = _router(xf, Wr)
    p0f = pos0.reshape(N)
    p1f = pos1.reshape(N)
    xg = _make_dispatch()(xpk, p0f, p1f)
    yg = _grouped(te.reshape(64), xg, W1, W2, W3)
    sel0, sel1 = _make_gather2()(yg, p0f, p1f)
    out = _final(xpk, Ws1, Ws2, Ws3, sel0, sel1, w0, w1)
    return out.reshape(orig_shape)
